# baseline (speedup 1.0000x reference)
import functools

import jax
import jax.numpy as jnp
from jax import lax
from jax.experimental import pallas as pl
from jax.experimental.pallas import tpu as pltpu
from jax.experimental.pallas import tpu_sc as plsc

NC = 2
NS = 16
NW = NC * NS
SEQ = 8192
TOK = SEQ // NW
CH = 128
NCH = TOK // CH
D = 50


def _sc_partials(x_hbm, emb_hbm, part_hbm, idx_v, rows_v, out_v, sem):
    wid = lax.axis_index("s") * NC + lax.axis_index("c")

    pltpu.sync_copy(x_hbm.at[pl.ds(wid * TOK, TOK)], idx_v)

    def fire(c, _):
        vec = idx_v[pl.ds(c * 16, 16)]
        v0 = vec[0]
        for k in range(1):
            pltpu.async_copy(
                emb_hbm.at[pl.ds(v0, 1)],
                rows_v.at[pl.ds(c * 16 + k, 1)],
                sem,
            )
        return 0

    lax.fori_loop(0, TOK // 16, fire, 0)
    pltpu.make_async_copy(emb_hbm.at[pl.ds(0, 16)], rows_v.at[pl.ds(0, 16)], sem).wait()

    lane = lax.iota(jnp.int32, 16)
    zero = jnp.zeros((16,), jnp.float32)
    one = jnp.ones((16,), jnp.float32)

    def body(i, carry):
        s0, s1, s2, s3, c0, c1, c2, c3 = carry
        r0 = rows_v[i, pl.ds(0, 16)]
        r1 = rows_v[i, pl.ds(16, 16)]
        r2 = rows_v[i, pl.ds(32, 16)]
        r3 = jnp.where(lane >= 14, rows_v[i, pl.ds(34, 16)], zero)
        s0 = s0 + r0
        s1 = s1 + r1
        s2 = s2 + r2
        s3 = s3 + r3
        c0 = c0 + jnp.where(r0 != 0.0, one, zero)
        c1 = c1 + jnp.where(r1 != 0.0, one, zero)
        c2 = c2 + jnp.where(r2 != 0.0, one, zero)
        c3 = c3 + jnp.where(r3 != 0.0, one, zero)
        return s0, s1, s2, s3, c0, c1, c2, c3

    acc = lax.fori_loop(0, TOK, body, (zero,) * 8)
    for p in range(8):
        out_v[pl.ds(p * 16, 16)] = acc[p]
    pltpu.sync_copy(out_v, part_hbm.at[wid])


@jax.jit
def _stage1(x1d, emb):
    mesh = plsc.VectorSubcoreMesh(core_axis_name="c", subcore_axis_name="s")
    f = pl.kernel(
        _sc_partials,
        out_type=jax.ShapeDtypeStruct((NW, 128), jnp.float32),
        mesh=mesh,
        scratch_types=[
            pltpu.VMEM((TOK,), jnp.int32),
            pltpu.VMEM((TOK, D), jnp.float32),
            pltpu.VMEM((128,), jnp.float32),
            pltpu.SemaphoreType.DMA,
        ],
    )
    return f(x1d, emb)


def _tc_epilogue(part_ref, w64_ref, b8_ref, eye_ref, out_ref):
    total = jnp.sum(part_ref[...], axis=0, keepdims=True)
    sums = total[:, :64]
    cnts = total[:, 64:]
    sw = lax.dot_general(sums, w64_ref[...],
                         (((1,), (1,)), ((), ())),
                         preferred_element_type=jnp.float32)
    denom = cnts + jnp.where(cnts == 0.0, 1e-10, 0.0)
    recip = 1.0 / denom
    recip_col = lax.dot_general(eye_ref[...], recip,
                                (((1,), (1,)), ((), ())),
                                preferred_element_type=jnp.float32)
    out_ref[...] = recip_col * sw + b8_ref[...]


@jax.jit
def _stage2(part, w64, b8, eye):
    return pl.pallas_call(
        _tc_epilogue,
        out_shape=jax.ShapeDtypeStruct((64, 8), jnp.float32),
    )(part, w64, b8, eye)


def kernel(x, emb, W, b):
    part = _stage1(x.reshape(SEQ), emb)

    w64 = jnp.zeros((8, 64), jnp.float32)
    w64 = w64.at[:3, :48].set(W[:, :48])
    w64 = w64.at[:3, 62].set(W[:, 48])
    w64 = w64.at[:3, 63].set(W[:, 49])
    b8 = jnp.zeros((1, 8), jnp.float32).at[0, :3].set(b)
    eye = jnp.eye(64, dtype=jnp.float32)

    y64 = _stage2(part, w64, b8, eye)
    y = jnp.concatenate([y64[:48, :3], y64[62:64, :3]], axis=0)
    return y[None]


# speedup vs baseline: 2.0038x; 2.0038x over previous
import jax
import jax.numpy as jnp
from jax import lax
from jax.experimental import pallas as pl
from jax.experimental.pallas import tpu as pltpu
from jax.experimental.pallas import tpu_sc as plsc

NC = 2
NS = 16
NW = NC * NS
SEQ = 8192
TOK = SEQ // NW
D = 50
V = 1000000
BLK = 8192
NBLK = (V + BLK - 1) // BLK
VP = NBLK * BLK
HSLC = VP // NS
ZB = HSLC // 8


def _sc_hist(x_hbm, hist_hbm, idx_v, ones_v, zero_v, hist_s, sem):
    cid = lax.axis_index("c")
    sid = lax.axis_index("s")
    wid = sid * NC + cid

    def zfill(i, _):
        zero_v[pl.ds(i * 16, 16)] = jnp.zeros((16,), jnp.float32)
        return 0

    lax.fori_loop(0, ZB // 16, zfill, 0)
    for r in range(8):
        pltpu.sync_copy(zero_v, hist_s.at[pl.ds(sid * HSLC + r * ZB, ZB)])

    for j in range(TOK // 128):
        pltpu.sync_copy(x_hbm.at[pl.ds(wid * TOK + j * 128, 128)], idx_v.at[j])

    def ofill(i, _):
        ones_v[pl.ds(i * 16, 16)] = jnp.ones((16,), jnp.float32)
        return 0

    lax.fori_loop(0, 128 // 16, ofill, 0)
    plsc.subcore_barrier()
    for j in range(TOK // 128):
        pltpu.sync_copy(ones_v, hist_s.at[idx_v.at[j]], add=True)
    plsc.subcore_barrier()

    pltpu.sync_copy(
        hist_s.at[pl.ds(sid * HSLC, HSLC)],
        hist_hbm.at[pl.ds(cid * VP + sid * HSLC, HSLC)],
    )


@jax.jit
def _stage1(x1d):
    mesh = plsc.VectorSubcoreMesh(core_axis_name="c", subcore_axis_name="s")
    f = pl.kernel(
        _sc_hist,
        out_type=jax.ShapeDtypeStruct((NC * VP,), jnp.float32),
        mesh=mesh,
        scratch_types=[
            pltpu.VMEM((TOK // 128, 128), jnp.int32),
            pltpu.VMEM((128,), jnp.float32),
            pltpu.VMEM((ZB,), jnp.float32),
            pltpu.VMEM_SHARED((VP,), jnp.float32),
            pltpu.SemaphoreType.DMA,
        ],
    )
    return f(x1d)


def _tc_contract(embt_ref, hist0_ref, hist1_ref, out_ref):
    i = pl.program_id(0)

    @pl.when(i == 0)
    def _():
        out_ref[...] = jnp.zeros_like(out_ref)

    col = i * BLK + lax.broadcasted_iota(jnp.int32, (D, BLK), 1)
    e = jnp.where(col < V, embt_ref[...], 0.0)
    h = (hist0_ref[...] + hist1_ref[...]).reshape(1, BLK)
    m = jnp.where((e != 0.0) & (col < V), 1.0, 0.0)
    s_col = lax.dot_general(e, h, (((1,), (1,)), ((), ())),
                            preferred_element_type=jnp.float32)
    c_col = lax.dot_general(m, h, (((1,), (1,)), ((), ())),
                            preferred_element_type=jnp.float32)
    onehot0 = (lax.broadcasted_iota(jnp.int32, (1, 8), 1) == 0).astype(jnp.float32)
    onehot1 = (lax.broadcasted_iota(jnp.int32, (1, 8), 1) == 1).astype(jnp.float32)
    out_ref[...] += (
        lax.dot_general(s_col, onehot0, (((1,), (0,)), ((), ())),
                        preferred_element_type=jnp.float32)
        + lax.dot_general(c_col, onehot1, (((1,), (0,)), ((), ())),
                          preferred_element_type=jnp.float32)
    )


@jax.jit
def _stage2(embt, hist):
    return pl.pallas_call(
        _tc_contract,
        grid=(NBLK,),
        in_specs=[
            pl.BlockSpec((D, BLK), lambda i: (0, i)),
            pl.BlockSpec((BLK,), lambda i: (i,)),
            pl.BlockSpec((BLK,), lambda i: (NBLK + i,)),
        ],
        out_specs=pl.BlockSpec((D, 8), lambda i: (0, 0)),
        out_shape=jax.ShapeDtypeStruct((D, 8), jnp.float32),
    )(embt, hist, hist)


def _tc_epilogue(sc_ref, w8_ref, b8_ref, out_ref):
    s_col = sc_ref[:, 0:1]
    ms_col = sc_ref[:, 1:2]
    sw = lax.dot_general(s_col, w8_ref[...],
                         (((0,), (1,)), ((), ())),
                         preferred_element_type=jnp.float32)
    denom = ms_col + jnp.where(ms_col == 0.0, 1e-10, 0.0)
    out_ref[...] = (1.0 / denom) * sw + b8_ref[...]


@jax.jit
def _stage3(sc, w8, b8):
    return pl.pallas_call(
        _tc_epilogue,
        out_shape=jax.ShapeDtypeStruct((D, 8), jnp.float32),
    )(sc, w8, b8)


def kernel(x, emb, W, b):
    hist = _stage1(x.reshape(SEQ))
    embt = jnp.swapaxes(emb, 0, 1)
    sc = _stage2(embt, hist)
    w8 = jnp.zeros((8, D), jnp.float32).at[:3].set(W)
    b8 = jnp.zeros((1, 8), jnp.float32).at[0, :3].set(b)
    y = _stage3(sc, w8, b8)
    return y[:, :3][None]


# speedup vs baseline: 2.1115x; 1.0537x over previous
import jax
import jax.numpy as jnp
from jax import lax
from jax.experimental import pallas as pl
from jax.experimental.pallas import tpu as pltpu
from jax.experimental.pallas import tpu_sc as plsc

NC = 2
NS = 16
NW = NC * NS
SEQ = 8192
TOK = SEQ // NW
D = 50
V = 1000000
BLK = 8192
NBLK = (V + BLK - 1) // BLK
VP = NBLK * BLK
HSLC = VP // NS
ZB = HSLC // 8


def _sc_hist(x_hbm, hist_hbm, idx_v, ones_v, zero_v, hist_s, sem, zsem):
    cid = lax.axis_index("c")
    sid = lax.axis_index("s")
    wid = sid * NC + cid

    def zfill(i, _):
        zero_v[pl.ds(i * 16, 16)] = jnp.zeros((16,), jnp.float32)
        return 0

    for j in range(TOK // 128):
        pltpu.async_copy(x_hbm.at[pl.ds(wid * TOK + j * 128, 128)], idx_v.at[j], sem)

    lax.fori_loop(0, ZB // 16, zfill, 0)

    def ofill(i, _):
        ones_v[pl.ds(i * 16, 16)] = jnp.ones((16,), jnp.float32)
        return 0

    lax.fori_loop(0, 128 // 16, ofill, 0)
    for r in range(8):
        pltpu.async_copy(zero_v, hist_s.at[pl.ds(sid * HSLC + r * ZB, ZB)], zsem)
    for j in range(TOK // 128):
        pltpu.make_async_copy(x_hbm.at[pl.ds(wid * TOK + j * 128, 128)], idx_v.at[j], sem).wait()
    for r in range(8):
        pltpu.make_async_copy(zero_v, hist_s.at[pl.ds(sid * HSLC + r * ZB, ZB)], zsem).wait()
    plsc.subcore_barrier()
    for j in range(TOK // 128):
        pltpu.sync_copy(ones_v, hist_s.at[idx_v.at[j]], add=True)
    plsc.subcore_barrier()

    pltpu.sync_copy(
        hist_s.at[pl.ds(sid * HSLC, HSLC)],
        hist_hbm.at[pl.ds(cid * VP + sid * HSLC, HSLC)],
    )


@jax.jit
def _stage1(x1d):
    mesh = plsc.VectorSubcoreMesh(core_axis_name="c", subcore_axis_name="s")
    f = pl.kernel(
        _sc_hist,
        out_type=jax.ShapeDtypeStruct((NC * VP,), jnp.float32),
        mesh=mesh,
        scratch_types=[
            pltpu.VMEM((TOK // 128, 128), jnp.int32),
            pltpu.VMEM((128,), jnp.float32),
            pltpu.VMEM((ZB,), jnp.float32),
            pltpu.VMEM_SHARED((VP,), jnp.float32),
            pltpu.SemaphoreType.DMA,
            pltpu.SemaphoreType.DMA,
        ],
    )
    return f(x1d)


def _tc_contract(embt_ref, hist0_ref, hist1_ref, out_ref):
    i = pl.program_id(0)

    @pl.when(i == 0)
    def _():
        out_ref[...] = jnp.zeros_like(out_ref)

    h = (hist0_ref[...] + hist1_ref[...]).reshape(1, BLK)

    def accumulate(e):
        m = jnp.where(e != 0.0, 1.0, 0.0)
        s_col = lax.dot_general(e, h, (((1,), (1,)), ((), ())),
                                preferred_element_type=jnp.float32)
        c_col = lax.dot_general(m, h, (((1,), (1,)), ((), ())),
                                preferred_element_type=jnp.float32)
        onehot0 = (lax.broadcasted_iota(jnp.int32, (1, 8), 1) == 0)
        onehot1 = (lax.broadcasted_iota(jnp.int32, (1, 8), 1) == 1)
        out_ref[...] += (
            lax.dot_general(s_col, onehot0.astype(jnp.float32),
                            (((1,), (0,)), ((), ())),
                            preferred_element_type=jnp.float32)
            + lax.dot_general(c_col, onehot1.astype(jnp.float32),
                              (((1,), (0,)), ((), ())),
                              preferred_element_type=jnp.float32)
        )

    @pl.when(i < NBLK - 1)
    def _():
        accumulate(embt_ref[...])

    @pl.when(i == NBLK - 1)
    def _():
        col = lax.broadcasted_iota(jnp.int32, (D, BLK), 1)
        accumulate(jnp.where(col < V - (NBLK - 1) * BLK, embt_ref[...], 0.0))


@jax.jit
def _stage2(embt, hist):
    return pl.pallas_call(
        _tc_contract,
        grid=(NBLK,),
        in_specs=[
            pl.BlockSpec((D, BLK), lambda i: (0, i)),
            pl.BlockSpec((BLK,), lambda i: (i,)),
            pl.BlockSpec((BLK,), lambda i: (NBLK + i,)),
        ],
        out_specs=pl.BlockSpec((D, 8), lambda i: (0, 0)),
        out_shape=jax.ShapeDtypeStruct((D, 8), jnp.float32),
    )(embt, hist, hist)


def _tc_epilogue(sc_ref, w8_ref, b8_ref, out_ref):
    s_col = sc_ref[:, 0:1]
    ms_col = sc_ref[:, 1:2]
    sw = lax.dot_general(s_col, w8_ref[...],
                         (((0,), (1,)), ((), ())),
                         preferred_element_type=jnp.float32)
    denom = ms_col + jnp.where(ms_col == 0.0, 1e-10, 0.0)
    out_ref[...] = (1.0 / denom) * sw + b8_ref[...]


@jax.jit
def _stage3(sc, w8, b8):
    return pl.pallas_call(
        _tc_epilogue,
        out_shape=jax.ShapeDtypeStruct((D, 8), jnp.float32),
    )(sc, w8, b8)


def kernel(x, emb, W, b):
    hist = _stage1(x.reshape(SEQ))
    embt = jnp.swapaxes(emb, 0, 1)
    sc = _stage2(embt, hist)
    w8 = jnp.zeros((8, D), jnp.float32).at[:3].set(W)
    b8 = jnp.zeros((1, 8), jnp.float32).at[0, :3].set(b)
    y = _stage3(sc, w8, b8)
    return y[:, :3][None]


# speedup vs baseline: 2.2379x; 1.0599x over previous
import jax
import jax.numpy as jnp
from jax import lax
from jax.experimental import pallas as pl
from jax.experimental.pallas import tpu as pltpu
from jax.experimental.pallas import tpu_sc as plsc

NC = 2
NS = 16
NW = NC * NS
SEQ = 8192
TOK = SEQ // NW
D = 50
V = 1000000
BLK = 8192
NBLK = (V + BLK - 1) // BLK
VP = NBLK * BLK
HSLC = VP // NS
ZB = HSLC // 8


def _sc_hist(x_hbm, hist_hbm, idx_v, ones_v, zero_v, hist_s, sem, zsem):
    cid = lax.axis_index("c")
    sid = lax.axis_index("s")
    wid = sid * NC + cid

    def zfill(i, _):
        zero_v[pl.ds(i * 16, 16)] = jnp.zeros((16,), jnp.float32)
        return 0

    for j in range(TOK // 128):
        pltpu.async_copy(x_hbm.at[pl.ds(wid * TOK + j * 128, 128)], idx_v.at[j], sem)

    lax.fori_loop(0, ZB // 16, zfill, 0)

    def ofill(i, _):
        ones_v[pl.ds(i * 16, 16)] = jnp.ones((16,), jnp.float32)
        return 0

    lax.fori_loop(0, 128 // 16, ofill, 0)
    for r in range(8):
        pltpu.async_copy(zero_v, hist_s.at[pl.ds(sid * HSLC + r * ZB, ZB)], zsem)
    for j in range(TOK // 128):
        pltpu.make_async_copy(x_hbm.at[pl.ds(wid * TOK + j * 128, 128)], idx_v.at[j], sem).wait()
    for r in range(8):
        pltpu.make_async_copy(zero_v, hist_s.at[pl.ds(sid * HSLC + r * ZB, ZB)], zsem).wait()
    plsc.subcore_barrier()
    for j in range(TOK // 128):
        pltpu.sync_copy(ones_v, hist_s.at[idx_v.at[j]], add=True)
    plsc.subcore_barrier()

    pltpu.sync_copy(
        hist_s.at[pl.ds(sid * HSLC, HSLC)],
        hist_hbm.at[pl.ds(cid * VP + sid * HSLC, HSLC)],
    )


@jax.jit
def _stage1(x1d):
    mesh = plsc.VectorSubcoreMesh(core_axis_name="c", subcore_axis_name="s")
    f = pl.kernel(
        _sc_hist,
        out_type=jax.ShapeDtypeStruct((NC * VP,), jnp.float32),
        mesh=mesh,
        scratch_types=[
            pltpu.VMEM((TOK // 128, 128), jnp.int32),
            pltpu.VMEM((128,), jnp.float32),
            pltpu.VMEM((ZB,), jnp.float32),
            pltpu.VMEM_SHARED((VP,), jnp.float32),
            pltpu.SemaphoreType.DMA,
            pltpu.SemaphoreType.DMA,
        ],
    )
    return f(x1d)


def _tc_contract(embt_ref, hist0_ref, hist1_ref, s_ref, c_ref):
    i = pl.program_id(0)

    @pl.when(i == 0)
    def _():
        s_ref[...] = jnp.zeros_like(s_ref)
        c_ref[...] = jnp.zeros_like(c_ref)

    h = (hist0_ref[...] + hist1_ref[...]).reshape(1, BLK)

    def accumulate(e):
        eh = e * h
        mh = jnp.where(e != 0.0, h, 0.0)
        s_ref[...] += jnp.sum(eh, axis=1, keepdims=True)
        c_ref[...] += jnp.sum(mh, axis=1, keepdims=True)

    @pl.when(i < NBLK - 1)
    def _():
        accumulate(embt_ref[...])

    @pl.when(i == NBLK - 1)
    def _():
        col = lax.broadcasted_iota(jnp.int32, (D, BLK), 1)
        accumulate(jnp.where(col < V - (NBLK - 1) * BLK, embt_ref[...], 0.0))


@jax.jit
def _stage2(embt, hist):
    return pl.pallas_call(
        _tc_contract,
        grid=(NBLK,),
        in_specs=[
            pl.BlockSpec((D, BLK), lambda i: (0, i)),
            pl.BlockSpec((BLK,), lambda i: (i,)),
            pl.BlockSpec((BLK,), lambda i: (NBLK + i,)),
        ],
        out_specs=[
            pl.BlockSpec((D, 1), lambda i: (0, 0)),
            pl.BlockSpec((D, 1), lambda i: (0, 0)),
        ],
        out_shape=[
            jax.ShapeDtypeStruct((D, 1), jnp.float32),
            jax.ShapeDtypeStruct((D, 1), jnp.float32),
        ],
    )(embt, hist, hist)


def _tc_epilogue(s_ref, c_ref, w8_ref, b8_ref, out_ref):
    s_col = s_ref[...]
    ms_col = c_ref[...]
    sw = lax.dot_general(s_col, w8_ref[...],
                         (((0,), (1,)), ((), ())),
                         preferred_element_type=jnp.float32)
    denom = ms_col + jnp.where(ms_col == 0.0, 1e-10, 0.0)
    out_ref[...] = (1.0 / denom) * sw + b8_ref[...]


@jax.jit
def _stage3(s, c, w8, b8):
    return pl.pallas_call(
        _tc_epilogue,
        out_shape=jax.ShapeDtypeStruct((D, 8), jnp.float32),
    )(s, c, w8, b8)


def kernel(x, emb, W, b):
    hist = _stage1(x.reshape(SEQ))
    embt = jnp.swapaxes(emb, 0, 1)
    s, c = _stage2(embt, hist)
    w8 = jnp.zeros((8, D), jnp.float32).at[:3].set(W)
    b8 = jnp.zeros((1, 8), jnp.float32).at[0, :3].set(b)
    y = _stage3(s, c, w8, b8)
    return y[:, :3][None]


# speedup vs baseline: 2.8424x; 1.2701x over previous
import jax
import jax.numpy as jnp
from jax import lax
from jax.experimental import pallas as pl
from jax.experimental.pallas import tpu as pltpu
from jax.experimental.pallas import tpu_sc as plsc

NC = 2
NS = 16
NW = NC * NS
SEQ = 8192
TOK = SEQ // NW
D = 50
V = 1000000
BLK = 16384
NBLK = (V + BLK - 1) // BLK
VP = NBLK * BLK
HSLC = VP // NS
ZB = HSLC // 8


def _sc_hist(x_hbm, hist_hbm, idx_v, ones_v, zero_v, hist_s, sem, zsem):
    cid = lax.axis_index("c")
    sid = lax.axis_index("s")
    wid = sid * NC + cid

    def zfill(i, _):
        zero_v[pl.ds(i * 16, 16)] = jnp.zeros((16,), jnp.float32)
        return 0

    for j in range(TOK // 128):
        pltpu.async_copy(x_hbm.at[pl.ds(wid * TOK + j * 128, 128)], idx_v.at[j], sem)

    lax.fori_loop(0, ZB // 16, zfill, 0)

    def ofill(i, _):
        ones_v[pl.ds(i * 16, 16)] = jnp.ones((16,), jnp.float32)
        return 0

    lax.fori_loop(0, 128 // 16, ofill, 0)
    for r in range(8):
        pltpu.async_copy(zero_v, hist_s.at[pl.ds(sid * HSLC + r * ZB, ZB)], zsem)
    for j in range(TOK // 128):
        pltpu.make_async_copy(x_hbm.at[pl.ds(wid * TOK + j * 128, 128)], idx_v.at[j], sem).wait()
    for r in range(8):
        pltpu.make_async_copy(zero_v, hist_s.at[pl.ds(sid * HSLC + r * ZB, ZB)], zsem).wait()
    plsc.subcore_barrier()
    for j in range(TOK // 128):
        pltpu.sync_copy(ones_v, hist_s.at[idx_v.at[j]], add=True)
    plsc.subcore_barrier()

    pltpu.sync_copy(
        hist_s.at[pl.ds(sid * HSLC, HSLC)],
        hist_hbm.at[pl.ds(cid * VP + sid * HSLC, HSLC)],
    )


@jax.jit
def _stage1(x1d):
    mesh = plsc.VectorSubcoreMesh(core_axis_name="c", subcore_axis_name="s")
    f = pl.kernel(
        _sc_hist,
        out_type=jax.ShapeDtypeStruct((NC * VP,), jnp.float32),
        mesh=mesh,
        scratch_types=[
            pltpu.VMEM((TOK // 128, 128), jnp.int32),
            pltpu.VMEM((128,), jnp.float32),
            pltpu.VMEM((ZB,), jnp.float32),
            pltpu.VMEM_SHARED((VP,), jnp.float32),
            pltpu.SemaphoreType.DMA,
            pltpu.SemaphoreType.DMA,
        ],
    )
    return f(x1d)


def _tc_contract(embt_ref, hist0_ref, hist1_ref, w8_ref, b8_ref, out_ref,
                 s_ref, c_ref):
    i = pl.program_id(0)

    @pl.when(i == 0)
    def _():
        s_ref[...] = jnp.zeros_like(s_ref)
        c_ref[...] = jnp.zeros_like(c_ref)

    h = (hist0_ref[...] + hist1_ref[...]).reshape(1, BLK)

    def accumulate(e):
        eh = e * h
        mh = jnp.where(e != 0.0, h, 0.0)
        s_ref[...] += jnp.sum(eh, axis=1, keepdims=True)
        c_ref[...] += jnp.sum(mh, axis=1, keepdims=True)

    @pl.when(i < NBLK - 1)
    def _():
        accumulate(embt_ref[...])

    @pl.when(i == NBLK - 1)
    def _():
        col = lax.broadcasted_iota(jnp.int32, (D, BLK), 1)
        accumulate(jnp.where(col < V - (NBLK - 1) * BLK, embt_ref[...], 0.0))
        s_col = s_ref[...]
        ms_col = c_ref[...]
        sw = lax.dot_general(s_col, w8_ref[...],
                             (((0,), (1,)), ((), ())),
                             preferred_element_type=jnp.float32)
        denom = ms_col + jnp.where(ms_col == 0.0, 1e-10, 0.0)
        out_ref[...] = (1.0 / denom) * sw + b8_ref[...]


@jax.jit
def _stage2(embt, hist, w8, b8):
    return pl.pallas_call(
        _tc_contract,
        grid=(NBLK,),
        in_specs=[
            pl.BlockSpec((D, BLK), lambda i: (0, i)),
            pl.BlockSpec((BLK,), lambda i: (i,)),
            pl.BlockSpec((BLK,), lambda i: (NBLK + i,)),
            pl.BlockSpec((8, D), lambda i: (0, 0)),
            pl.BlockSpec((1, 8), lambda i: (0, 0)),
        ],
        out_specs=pl.BlockSpec((D, 8), lambda i: (0, 0)),
        out_shape=jax.ShapeDtypeStruct((D, 8), jnp.float32),
        scratch_shapes=[
            pltpu.VMEM((D, 1), jnp.float32),
            pltpu.VMEM((D, 1), jnp.float32),
        ],
    )(embt, hist, hist, w8, b8)


def kernel(x, emb, W, b):
    hist = _stage1(x.reshape(SEQ))
    embt = jnp.swapaxes(emb, 0, 1)
    w8 = jnp.zeros((8, D), jnp.float32).at[:3].set(W)
    b8 = jnp.zeros((1, 8), jnp.float32).at[0, :3].set(b)
    y = _stage2(embt, hist, w8, b8)
    return y[:, :3][None]


# speedup vs baseline: 3.2420x; 1.1406x over previous
import jax
import jax.numpy as jnp
from jax import lax
from jax.experimental import pallas as pl
from jax.experimental.pallas import tpu as pltpu
from jax.experimental.pallas import tpu_sc as plsc

NC = 2
NS = 16
NW = NC * NS
SEQ = 8192
TOK = SEQ // NW
D = 50
V = 1000000
BLK = 32768
NBLK = (V + BLK - 1) // BLK
VP = NBLK * BLK
HSLC = VP // NS
ZB = HSLC // 8


def _sc_hist(x_hbm, hist_hbm, idx_v, ones_v, zero_v, hist_s, sem, zsem):
    cid = lax.axis_index("c")
    sid = lax.axis_index("s")
    wid = sid * NC + cid

    def zfill(i, _):
        zero_v[pl.ds(i * 16, 16)] = jnp.zeros((16,), jnp.float32)
        return 0

    for j in range(TOK // 128):
        pltpu.async_copy(x_hbm.at[pl.ds(wid * TOK + j * 128, 128)], idx_v.at[j], sem)

    lax.fori_loop(0, ZB // 16, zfill, 0)

    def ofill(i, _):
        ones_v[pl.ds(i * 16, 16)] = jnp.ones((16,), jnp.float32)
        return 0

    lax.fori_loop(0, 128 // 16, ofill, 0)
    for r in range(8):
        pltpu.async_copy(zero_v, hist_s.at[pl.ds(sid * HSLC + r * ZB, ZB)], zsem)
    for j in range(TOK // 128):
        pltpu.make_async_copy(x_hbm.at[pl.ds(wid * TOK + j * 128, 128)], idx_v.at[j], sem).wait()
    for r in range(8):
        pltpu.make_async_copy(zero_v, hist_s.at[pl.ds(sid * HSLC + r * ZB, ZB)], zsem).wait()
    plsc.subcore_barrier()
    for j in range(TOK // 128):
        pltpu.sync_copy(ones_v, hist_s.at[idx_v.at[j]], add=True)
    plsc.subcore_barrier()

    pltpu.sync_copy(
        hist_s.at[pl.ds(sid * HSLC, HSLC)],
        hist_hbm.at[pl.ds(cid * VP + sid * HSLC, HSLC)],
    )


@jax.jit
def _stage1(x1d):
    mesh = plsc.VectorSubcoreMesh(core_axis_name="c", subcore_axis_name="s")
    f = pl.kernel(
        _sc_hist,
        out_type=jax.ShapeDtypeStruct((NC * VP,), jnp.float32),
        mesh=mesh,
        scratch_types=[
            pltpu.VMEM((TOK // 128, 128), jnp.int32),
            pltpu.VMEM((128,), jnp.float32),
            pltpu.VMEM((ZB,), jnp.float32),
            pltpu.VMEM_SHARED((VP,), jnp.float32),
            pltpu.SemaphoreType.DMA,
            pltpu.SemaphoreType.DMA,
        ],
    )
    return f(x1d)


def _tc_contract(embt_ref, hist0_ref, hist1_ref, w8_ref, b8_ref, out_ref,
                 s_ref, c_ref):
    i = pl.program_id(0)

    @pl.when(i == 0)
    def _():
        s_ref[...] = jnp.zeros_like(s_ref)
        c_ref[...] = jnp.zeros_like(c_ref)

    h = (hist0_ref[...] + hist1_ref[...]).reshape(1, BLK)

    def accumulate(e):
        eh = e * h
        mh = jnp.where(e != 0.0, h, 0.0)
        s_ref[...] += jnp.sum(eh, axis=1, keepdims=True)
        c_ref[...] += jnp.sum(mh, axis=1, keepdims=True)

    @pl.when(i < NBLK - 1)
    def _():
        accumulate(embt_ref[...])

    @pl.when(i == NBLK - 1)
    def _():
        col = lax.broadcasted_iota(jnp.int32, (D, BLK), 1)
        accumulate(jnp.where(col < V - (NBLK - 1) * BLK, embt_ref[...], 0.0))
        s_col = s_ref[...]
        ms_col = c_ref[...]
        sw = lax.dot_general(s_col, w8_ref[...],
                             (((0,), (1,)), ((), ())),
                             preferred_element_type=jnp.float32)
        denom = ms_col + jnp.where(ms_col == 0.0, 1e-10, 0.0)
        out_ref[...] = (1.0 / denom) * sw + b8_ref[...]


@jax.jit
def _stage2(embt, hist, w8, b8):
    return pl.pallas_call(
        _tc_contract,
        grid=(NBLK,),
        in_specs=[
            pl.BlockSpec((D, BLK), lambda i: (0, i)),
            pl.BlockSpec((BLK,), lambda i: (i,)),
            pl.BlockSpec((BLK,), lambda i: (NBLK + i,)),
            pl.BlockSpec((8, D), lambda i: (0, 0)),
            pl.BlockSpec((1, 8), lambda i: (0, 0)),
        ],
        out_specs=pl.BlockSpec((D, 8), lambda i: (0, 0)),
        out_shape=jax.ShapeDtypeStruct((D, 8), jnp.float32),
        scratch_shapes=[
            pltpu.VMEM((D, 1), jnp.float32),
            pltpu.VMEM((D, 1), jnp.float32),
        ],
    )(embt, hist, hist, w8, b8)


def kernel(x, emb, W, b):
    hist = _stage1(x.reshape(SEQ))
    embt = jnp.swapaxes(emb, 0, 1)
    w8 = jnp.zeros((8, D), jnp.float32).at[:3].set(W)
    b8 = jnp.zeros((1, 8), jnp.float32).at[0, :3].set(b)
    y = _stage2(embt, hist, w8, b8)
    return y[:, :3][None]


# speedup vs baseline: 3.3616x; 1.0369x over previous
import jax
import jax.numpy as jnp
from jax import lax
from jax.experimental import pallas as pl
from jax.experimental.pallas import tpu as pltpu
from jax.experimental.pallas import tpu_sc as plsc

NC = 2
NS = 16
NW = NC * NS
SEQ = 8192
TOK = SEQ // NW
D = 50
V = 1000000
BLK = 65536
NBLK = (V + BLK - 1) // BLK
VP = NBLK * BLK
HSLC = VP // NS
ZB = HSLC // 8


def _sc_hist(x_hbm, hist_hbm, idx_v, ones_v, zero_v, hist_s, sem, zsem):
    cid = lax.axis_index("c")
    sid = lax.axis_index("s")
    wid = sid * NC + cid

    def zfill(i, _):
        zero_v[pl.ds(i * 16, 16)] = jnp.zeros((16,), jnp.float32)
        return 0

    for j in range(TOK // 128):
        pltpu.async_copy(x_hbm.at[pl.ds(wid * TOK + j * 128, 128)], idx_v.at[j], sem)

    lax.fori_loop(0, ZB // 16, zfill, 0)

    def ofill(i, _):
        ones_v[pl.ds(i * 16, 16)] = jnp.ones((16,), jnp.float32)
        return 0

    lax.fori_loop(0, 128 // 16, ofill, 0)
    for r in range(8):
        pltpu.async_copy(zero_v, hist_s.at[pl.ds(sid * HSLC + r * ZB, ZB)], zsem)
    for j in range(TOK // 128):
        pltpu.make_async_copy(x_hbm.at[pl.ds(wid * TOK + j * 128, 128)], idx_v.at[j], sem).wait()
    for r in range(8):
        pltpu.make_async_copy(zero_v, hist_s.at[pl.ds(sid * HSLC + r * ZB, ZB)], zsem).wait()
    plsc.subcore_barrier()
    for j in range(TOK // 128):
        pltpu.sync_copy(ones_v, hist_s.at[idx_v.at[j]], add=True)
    plsc.subcore_barrier()

    pltpu.sync_copy(
        hist_s.at[pl.ds(sid * HSLC, HSLC)],
        hist_hbm.at[pl.ds(cid * VP + sid * HSLC, HSLC)],
    )


@jax.jit
def _stage1(x1d):
    mesh = plsc.VectorSubcoreMesh(core_axis_name="c", subcore_axis_name="s")
    f = pl.kernel(
        _sc_hist,
        out_type=jax.ShapeDtypeStruct((NC * VP,), jnp.float32),
        mesh=mesh,
        scratch_types=[
            pltpu.VMEM((TOK // 128, 128), jnp.int32),
            pltpu.VMEM((128,), jnp.float32),
            pltpu.VMEM((ZB,), jnp.float32),
            pltpu.VMEM_SHARED((VP,), jnp.float32),
            pltpu.SemaphoreType.DMA,
            pltpu.SemaphoreType.DMA,
        ],
    )
    return f(x1d)


def _tc_contract(embt_ref, hist0_ref, hist1_ref, w8_ref, b8_ref, out_ref,
                 s_ref, c_ref):
    i = pl.program_id(0)

    @pl.when(i == 0)
    def _():
        s_ref[...] = jnp.zeros_like(s_ref)
        c_ref[...] = jnp.zeros_like(c_ref)

    h = (hist0_ref[...] + hist1_ref[...]).reshape(1, BLK)

    def accumulate(e):
        eh = e * h
        mh = jnp.where(e != 0.0, h, 0.0)
        s_ref[...] += jnp.sum(eh, axis=1, keepdims=True)
        c_ref[...] += jnp.sum(mh, axis=1, keepdims=True)

    @pl.when(i < NBLK - 1)
    def _():
        accumulate(embt_ref[...])

    @pl.when(i == NBLK - 1)
    def _():
        col = lax.broadcasted_iota(jnp.int32, (D, BLK), 1)
        accumulate(jnp.where(col < V - (NBLK - 1) * BLK, embt_ref[...], 0.0))
        s_col = s_ref[...]
        ms_col = c_ref[...]
        sw = lax.dot_general(s_col, w8_ref[...],
                             (((0,), (1,)), ((), ())),
                             preferred_element_type=jnp.float32)
        denom = ms_col + jnp.where(ms_col == 0.0, 1e-10, 0.0)
        out_ref[...] = (1.0 / denom) * sw + b8_ref[...]


@jax.jit
def _stage2(embt, hist, w8, b8):
    return pl.pallas_call(
        _tc_contract,
        grid=(NBLK,),
        in_specs=[
            pl.BlockSpec((D, BLK), lambda i: (0, i)),
            pl.BlockSpec((BLK,), lambda i: (i,)),
            pl.BlockSpec((BLK,), lambda i: (NBLK + i,)),
            pl.BlockSpec((8, D), lambda i: (0, 0)),
            pl.BlockSpec((1, 8), lambda i: (0, 0)),
        ],
        out_specs=pl.BlockSpec((D, 8), lambda i: (0, 0)),
        out_shape=jax.ShapeDtypeStruct((D, 8), jnp.float32),
        scratch_shapes=[
            pltpu.VMEM((D, 1), jnp.float32),
            pltpu.VMEM((D, 1), jnp.float32),
        ],
    )(embt, hist, hist, w8, b8)


def kernel(x, emb, W, b):
    hist = _stage1(x.reshape(SEQ))
    embt = jnp.swapaxes(emb, 0, 1)
    w8 = jnp.zeros((8, D), jnp.float32).at[:3].set(W)
    b8 = jnp.zeros((1, 8), jnp.float32).at[0, :3].set(b)
    y = _stage2(embt, hist, w8, b8)
    return y[:, :3][None]
